# baseline (speedup 1.0000x reference)
import jax
import jax.numpy as jnp
from jax import lax
from jax.experimental import pallas as pl
from jax.experimental.pallas import tpu as pltpu, tpu_sc as plsc

D_MODEL = 256
HALF = 128
TABLE_COLS = 512
BATCH = 16384

_info = plsc.get_sparse_core_info()
NUM_CORES = _info.num_cores
NUM_SUBCORES = _info.num_subcores
NUM_WORKERS = NUM_CORES * NUM_SUBCORES
ROWS_PER_WORKER = BATCH // NUM_WORKERS
CHUNK = 64
CHUNKS_PER_WORKER = ROWS_PER_WORKER // CHUNK
LANES = 16
NBUF = 2


def _sc_body(x_hbm, coords_hbm, tab_hbm, out_hbm,
             tab_v, coords_v, xb0, xb1, sx0, sx1, so0, so1):
    wid = lax.axis_index("s") * NUM_CORES + lax.axis_index("c")
    base = wid * ROWS_PER_WORKER

    pltpu.sync_copy(tab_hbm, tab_v)
    pltpu.sync_copy(coords_hbm.at[pl.ds(base, ROWS_PER_WORKER)], coords_v)

    xb = (xb0, xb1)
    sx = (sx0, sx1)
    so = (so0, so1)
    iota = lax.iota(jnp.int32, LANES)

    def compute(j, b):
        def grp(g, _):
            rloc = g * LANES + iota
            rw = j * CHUNK + rloc
            col3 = jnp.full((LANES,), 3, jnp.int32)
            col2 = jnp.full((LANES,), 2, jnp.int32)
            wv = plsc.load_gather(coords_v, [rw, col3])
            hv = plsc.load_gather(coords_v, [rw, col2])

            @plsc.parallel_loop(0, HALF, unroll=8)
            def chan(c):
                cv = jnp.full((LANES,), 0, jnp.int32) + c
                tw = plsc.load_gather(tab_v, [cv, wv])
                xw = plsc.load_gather(xb[b], [rloc, cv])
                plsc.store_scatter(xb[b], [rloc, cv], xw + tw)
                th = plsc.load_gather(tab_v, [cv, hv])
                cv2 = cv + HALF
                xh = plsc.load_gather(xb[b], [rloc, cv2])
                plsc.store_scatter(xb[b], [rloc, cv2], xh + th)

            return 0

        lax.fori_loop(0, CHUNK // LANES, grp, 0)

    def outer(it, _):
        jj = it * NBUF
        descs = []
        for b in range(NBUF):
            @pl.when(it > 0)
            def _drain(b=b):
                pltpu.make_async_copy(
                    xb[b], out_hbm.at[pl.ds(0, CHUNK)], so[b]).wait()
            descs.append(pltpu.async_copy(
                x_hbm.at[pl.ds(base + (jj + b) * CHUNK, CHUNK)],
                xb[b], sx[b]))
        for b in range(NBUF):
            descs[b].wait()
            compute(jj + b, b)
            pltpu.async_copy(
                xb[b], out_hbm.at[pl.ds(base + (jj + b) * CHUNK, CHUNK)],
                so[b])
        return 0

    lax.fori_loop(0, CHUNKS_PER_WORKER // NBUF, outer, 0)
    for b in range(NBUF):
        pltpu.make_async_copy(
            xb[b], out_hbm.at[pl.ds(0, CHUNK)], so[b]).wait()


@jax.jit
def _pos_encode_add(x, coords, pos_table):
    tab = pos_table[:HALF, 0, :]
    mesh = plsc.VectorSubcoreMesh(core_axis_name="c", subcore_axis_name="s")
    run = pl.kernel(
        _sc_body,
        out_type=jax.ShapeDtypeStruct((BATCH, D_MODEL), jnp.float32),
        mesh=mesh,
        compiler_params=pltpu.CompilerParams(
            needs_layout_passes=False, use_tc_tiling_on_sc=False),
        scratch_types=[
            pltpu.VMEM((HALF, TABLE_COLS), jnp.float32),
            pltpu.VMEM((ROWS_PER_WORKER, 4), jnp.int32),
            pltpu.VMEM((CHUNK, D_MODEL), jnp.float32),
            pltpu.VMEM((CHUNK, D_MODEL), jnp.float32),
        ] + [pltpu.SemaphoreType.DMA] * 4,
    )
    return run(x, coords, tab)


def kernel(x, coords, pos_table):
    return _pos_encode_add(x, coords, pos_table)


# speedup vs baseline: 1.8337x; 1.8337x over previous
import jax
import jax.numpy as jnp
from jax import lax
from jax.experimental import pallas as pl
from jax.experimental.pallas import tpu as pltpu, tpu_sc as plsc

D_MODEL = 256
HALF = 128
TABLE_ROWS = 512
BATCH = 16384

_info = plsc.get_sparse_core_info()
NUM_CORES = _info.num_cores
NUM_SUBCORES = _info.num_subcores
NUM_WORKERS = NUM_CORES * NUM_SUBCORES
ROWS_PER_WORKER = BATCH // NUM_WORKERS
CHUNK = 64
CHUNKS_PER_WORKER = ROWS_PER_WORKER // CHUNK
LANES = 16
NBUF = 3


def _sc_body(x_hbm, coords_hbm, tab_hbm, out_hbm,
             coords_v, widx_v, hidx_v, xb, wr, hr, *sems):
    wid = lax.axis_index("s") * NUM_CORES + lax.axis_index("c")
    base = wid * ROWS_PER_WORKER

    pltpu.sync_copy(coords_hbm.at[pl.ds(base, ROWS_PER_WORKER)], coords_v)

    iota = lax.iota(jnp.int32, LANES)
    col3 = jnp.full((LANES,), 3, jnp.int32)
    col2 = jnp.full((LANES,), 2, jnp.int32)

    def bld(g, _):
        rw = g * LANES + iota
        widx_v[pl.ds(g * LANES, LANES)] = plsc.load_gather(
            coords_v, [rw, col3])
        hidx_v[pl.ds(g * LANES, LANES)] = plsc.load_gather(
            coords_v, [rw, col2])
        return 0

    lax.fori_loop(0, ROWS_PER_WORKER // LANES, bld, 0)

    sx = sems[0:NBUF]
    sw = sems[NBUF:2 * NBUF]
    sh = sems[2 * NBUF:3 * NBUF]
    so = sems[3 * NBUF:4 * NBUF]

    def issue(j):
        b = j % NBUF
        return (
            pltpu.async_copy(
                x_hbm.at[pl.ds(base + j * CHUNK, CHUNK)], xb.at[b], sx[b]),
            pltpu.async_copy(
                tab_hbm.at[widx_v.at[pl.ds(j * CHUNK, CHUNK)]],
                wr.at[b], sw[b]),
            pltpu.async_copy(
                tab_hbm.at[hidx_v.at[pl.ds(j * CHUNK, CHUNK)]],
                hr.at[b], sh[b]),
        )

    descs = [None] * CHUNKS_PER_WORKER
    outd = [None] * NBUF
    descs[0] = issue(0)
    descs[1] = issue(1)
    for j in range(CHUNKS_PER_WORKER):
        b = j % NBUF
        if j + 2 < CHUNKS_PER_WORKER:
            nb = (j + 2) % NBUF
            if outd[nb] is not None:
                outd[nb].wait()
            descs[j + 2] = issue(j + 2)
        for d in descs[j]:
            d.wait()

        @plsc.parallel_loop(0, CHUNK, unroll=8)
        def row_body(r):
            for t in range(HALF // LANES):
                o = t * LANES
                xb[b, r, pl.ds(o, LANES)] = (
                    xb[b, r, pl.ds(o, LANES)] + wr[b, r, pl.ds(o, LANES)])
                xb[b, r, pl.ds(HALF + o, LANES)] = (
                    xb[b, r, pl.ds(HALF + o, LANES)]
                    + hr[b, r, pl.ds(o, LANES)])

        outd[b] = pltpu.async_copy(
            xb.at[b], out_hbm.at[pl.ds(base + j * CHUNK, CHUNK)], so[b])
    for d in outd:
        if d is not None:
            d.wait()


@jax.jit
def _pos_encode_add(x, coords, pos_table):
    tab = jnp.transpose(pos_table[:HALF, 0, :])
    mesh = plsc.VectorSubcoreMesh(core_axis_name="c", subcore_axis_name="s")
    run = pl.kernel(
        _sc_body,
        out_type=jax.ShapeDtypeStruct((BATCH, D_MODEL), jnp.float32),
        mesh=mesh,
        compiler_params=pltpu.CompilerParams(
            needs_layout_passes=False, use_tc_tiling_on_sc=False),
        scratch_types=[
            pltpu.VMEM((ROWS_PER_WORKER, 4), jnp.int32),
            pltpu.VMEM((ROWS_PER_WORKER,), jnp.int32),
            pltpu.VMEM((ROWS_PER_WORKER,), jnp.int32),
            pltpu.VMEM((NBUF, CHUNK, D_MODEL), jnp.float32),
            pltpu.VMEM((NBUF, CHUNK, HALF), jnp.float32),
            pltpu.VMEM((NBUF, CHUNK, HALF), jnp.float32),
        ] + [pltpu.SemaphoreType.DMA] * (4 * NBUF),
    )
    return run(x, coords, tab)


def kernel(x, coords, pos_table):
    return _pos_encode_add(x, coords, pos_table)
